# baseline (speedup 1.0000x reference)
import functools

import jax
import jax.numpy as jnp
from jax import lax
from jax.experimental import pallas as pl
from jax.experimental.pallas import tpu as pltpu
from jax.experimental.pallas import tpu_sc as plsc

N = 10000
E = 320000
D = 128

NC = 2
NS = 16
NW = NC * NS

SPAD = 10240
ROWS_PER_SUB = SPAD // NS
EDGES_PER_SUB = E // NW
K = 16
NCHUNK = EDGES_PER_SUB // K


def _edge_body(t_hbm, c_hbm, src_hbm, dst_hbm, out_hbm,
               s_sh, ixs_all, ixd_all, ig0, ig1, ic0, ic1,
               r_t, r_c, h,
               semg0, semg1, semsc0, semsc1):
    cid = lax.axis_index("c")
    sid = lax.axis_index("s")
    wid = cid * NS + sid
    semg = (semg0, semg1)
    semsc = (semsc0, semsc1)
    igml = (ig0, ig1)
    icml = (ic0, ic1)

    def _zrow(i, _):
        for j in range(D // 16):
            h[0, i, pl.ds(j * 16, 16)] = jnp.zeros((16,), jnp.float32)
        return _
    lax.fori_loop(0, 2 * K, _zrow, None)

    def _zchunk(t, _):
        pltpu.sync_copy(
            h.at[0], s_sh.at[pl.ds(sid * ROWS_PER_SUB + t * 2 * K, 2 * K)])
        return _
    lax.fori_loop(0, ROWS_PER_SUB // (2 * K), _zchunk, None)

    ebase = wid * EDGES_PER_SUB
    pltpu.sync_copy(src_hbm.at[pl.ds(ebase, EDGES_PER_SUB)], ixs_all)
    pltpu.sync_copy(dst_hbm.at[pl.ds(ebase, EDGES_PER_SUB)], ixd_all)
    plsc.subcore_barrier()

    def _fill_idx(u, b):
        vs = ixs_all[pl.ds(u * K, K)]
        vd = ixd_all[pl.ds(u * K, K)]
        igml[b][pl.ds(0, K)] = vs
        igml[b][pl.ds(K, K)] = vd
        icml[b][pl.ds(0, K)] = vd
        icml[b][pl.ds(K, K)] = vs

    def _issue(u, b):
        pltpu.async_copy(t_hbm.at[igml[b]], r_t.at[b], semg[b])
        pltpu.async_copy(c_hbm.at[pl.ds(ebase + u * K, K)], r_c.at[b], semg[b])

    def _drain_g(b):
        pltpu.make_async_copy(t_hbm.at[pl.ds(0, 2 * K)], r_t.at[b], semg[b]).wait()
        pltpu.make_async_copy(c_hbm.at[pl.ds(0, K)], r_c.at[b], semg[b]).wait()

    def _unpk(w):
        lo = lax.bitcast_convert_type(jnp.left_shift(w, 16), jnp.float32)
        hi = lax.bitcast_convert_type(
            jnp.bitwise_and(w, jnp.int32(-65536)), jnp.float32)
        return lo, hi

    def _compute(b):
        @plsc.parallel_loop(0, K, unroll=2)
        def _row(i):
            for g in range(D // 32):
                sla = pl.ds(16 * g, 16)
                slb = pl.ds(D // 2 + 16 * g, 16)
                cl, ch = _unpk(r_c[b, i, sla])
                asl, ash = _unpk(r_t[b, i, sla])
                bsl, bsh = _unpk(r_t[b, i, slb])
                adl, adh = _unpk(r_t[b, K + i, sla])
                bdl, bdh = _unpk(r_t[b, K + i, slb])
                x1l = asl + bdl + cl
                x1h = ash + bdh + ch
                x2l = adl + bsl + cl
                x2h = adh + bsh + ch
                h[b, i, pl.ds(32 * g, 16)] = jnp.maximum(x1l, 0.01 * x1l)
                h[b, i, pl.ds(32 * g + 16, 16)] = jnp.maximum(x1h, 0.01 * x1h)
                h[b, K + i, pl.ds(32 * g, 16)] = jnp.maximum(x2l, 0.01 * x2l)
                h[b, K + i, pl.ds(32 * g + 16, 16)] = jnp.maximum(x2h, 0.01 * x2h)

    def _issue_sc(b):
        pltpu.async_copy(h.at[b], s_sh.at[icml[b]], semsc[b], add=True)

    def _drain_sc(b):
        pltpu.make_async_copy(h.at[b], s_sh.at[icml[b]], semsc[b]).wait()

    _fill_idx(0, 0)
    _issue(0, 0)

    @pl.loop(0, NCHUNK, step=2)
    def _ring(t):
        for b in range(2):
            u = t + b
            nb = (b + 1) % 2

            @pl.when(u < NCHUNK)
            def _():
                @pl.when(u + 1 < NCHUNK)
                def _():
                    @pl.when(u >= 1)
                    def _():
                        _drain_sc(nb)
                    _fill_idx(u + 1, nb)
                    _issue(u + 1, nb)
                _drain_g(b)
                _compute(b)
                _issue_sc(b)

    _drain_sc(0)
    _drain_sc(1)

    plsc.subcore_barrier()
    pltpu.sync_copy(s_sh.at[pl.ds(sid * ROWS_PER_SUB, ROWS_PER_SUB)],
                    out_hbm.at[cid, pl.ds(sid * ROWS_PER_SUB, ROWS_PER_SUB)])


@functools.lru_cache(maxsize=1)
def _edge_call():
    return pl.kernel(
        _edge_body,
        out_type=jax.ShapeDtypeStruct((NC, SPAD, D), jnp.float32),
        mesh=plsc.VectorSubcoreMesh(core_axis_name="c", subcore_axis_name="s"),
        scratch_types=[
        pltpu.VMEM_SHARED((SPAD, D), jnp.float32),
        pltpu.VMEM((EDGES_PER_SUB,), jnp.int32),
        pltpu.VMEM((EDGES_PER_SUB,), jnp.int32),
        pltpu.VMEM((2 * K,), jnp.int32),
        pltpu.VMEM((2 * K,), jnp.int32),
        pltpu.VMEM((2 * K,), jnp.int32),
        pltpu.VMEM((2 * K,), jnp.int32),
        pltpu.VMEM((2, 2 * K, D), jnp.int32),
        pltpu.VMEM((2, K, D // 2), jnp.int32),
        pltpu.VMEM((2, 2 * K, D), jnp.float32),
        pltpu.SemaphoreType.DMA,
        pltpu.SemaphoreType.DMA,
        pltpu.SemaphoreType.DMA,
        pltpu.SemaphoreType.DMA,
        ],
    )


def _rnd_pack(x):
    b = lax.bitcast_convert_type(x, jnp.int32)
    r = b + 0x7FFF + jnp.bitwise_and(lax.shift_right_logical(b, 16), 1)
    lo = lax.shift_right_logical(r[:, :D // 2], 16)
    hi = jnp.bitwise_and(r[:, D // 2:], jnp.int32(-65536))
    return jnp.bitwise_or(hi, lo)


def _prep_t_body(nf_ref, wa_ref, wb_ref, t_ref):
    x = nf_ref[...]
    t_ref[:, :D // 2] = _rnd_pack(x @ wa_ref[...])
    t_ref[:, D // 2:] = _rnd_pack(x @ wb_ref[...])


def _prep_c_body(ef_ref, wc_ref, c_ref):
    c_ref[...] = _rnd_pack(ef_ref[...] @ wc_ref[...])


def _node_body(sp_ref, nf_ref, we2_ref, wn1a_ref, wn1b_ref, wn2_ref, out_ref):
    red = (sp_ref[0] + sp_ref[1]) @ we2_ref[...]
    x = nf_ref[...] @ wn1a_ref[...] + red @ wn1b_ref[...]
    h = jnp.maximum(x, 0.01 * x)
    out_ref[...] = h @ wn2_ref[...]


_ROW_BLK = 400
_C_BLK = 2000


def kernel(nf, edge_index, ef, W_e1, W_e2, W_n1, W_n2):
    src = edge_index[0]
    dst = edge_index[1]

    T = pl.pallas_call(
        _prep_t_body,
        grid=(N // _ROW_BLK,),
        in_specs=[
            pl.BlockSpec((_ROW_BLK, D), lambda i: (i, 0)),
            pl.BlockSpec((D, D), lambda i: (0, 0)),
            pl.BlockSpec((D, D), lambda i: (0, 0)),
        ],
        out_specs=pl.BlockSpec((_ROW_BLK, D), lambda i: (i, 0)),
        out_shape=jax.ShapeDtypeStruct((N, D), jnp.int32),
    )(nf, W_e1[:D], W_e1[D:2 * D])

    C = pl.pallas_call(
        _prep_c_body,
        grid=(E // _C_BLK,),
        in_specs=[
            pl.BlockSpec((_C_BLK, 16), lambda i: (i, 0)),
            pl.BlockSpec((16, D), lambda i: (0, 0)),
        ],
        out_specs=pl.BlockSpec((_C_BLK, D // 2), lambda i: (i, 0)),
        out_shape=jax.ShapeDtypeStruct((E, D // 2), jnp.int32),
    )(ef, W_e1[2 * D:])

    s_parts = _edge_call()(T, C, src, dst)

    W_e2p = (W_e2.reshape(2, D // 32, 16, D)
             .transpose(1, 0, 2, 3).reshape(D, D))

    return pl.pallas_call(
        _node_body,
        grid=(N // _ROW_BLK,),
        in_specs=[
            pl.BlockSpec((NC, _ROW_BLK, D), lambda i: (0, i, 0)),
            pl.BlockSpec((_ROW_BLK, D), lambda i: (i, 0)),
            pl.BlockSpec((D, D), lambda i: (0, 0)),
            pl.BlockSpec((D, D), lambda i: (0, 0)),
            pl.BlockSpec((D, D), lambda i: (0, 0)),
            pl.BlockSpec((D, D), lambda i: (0, 0)),
        ],
        out_specs=pl.BlockSpec((_ROW_BLK, D), lambda i: (i, 0)),
        out_shape=jax.ShapeDtypeStruct((N, D), jnp.float32),
    )(s_parts, nf, W_e2p, W_n1[:D], W_n1[D:], W_n2)


# speedup vs baseline: 1.0080x; 1.0080x over previous
import functools

import jax
import jax.numpy as jnp
from jax import lax
from jax.experimental import pallas as pl
from jax.experimental.pallas import tpu as pltpu
from jax.experimental.pallas import tpu_sc as plsc

N = 10000
E = 320000
D = 128

NC = 2
NS = 16
NW = NC * NS

SPAD = 10240
ROWS_PER_SUB = SPAD // NS
EDGES_PER_SUB = E // NW
K = 16
NCHUNK = EDGES_PER_SUB // K


def _edge_body(t_hbm, c_hbm, src_hbm, dst_hbm, out_hbm,
               s_sh, ixs_all, ixd_all, ig0, ig1, ic0, ic1,
               r_t, r_c, h,
               semg0, semg1, semsc0, semsc1):
    cid = lax.axis_index("c")
    sid = lax.axis_index("s")
    wid = cid * NS + sid
    semg = (semg0, semg1)
    semsc = (semsc0, semsc1)
    igml = (ig0, ig1)
    icml = (ic0, ic1)

    def _zrow(i, _):
        for j in range(D // 16):
            h[0, i, pl.ds(j * 16, 16)] = jnp.zeros((16,), jnp.float32)
        return _
    lax.fori_loop(0, 2 * K, _zrow, None)

    def _zchunk(t, _):
        pltpu.sync_copy(
            h.at[0], s_sh.at[pl.ds(sid * ROWS_PER_SUB + t * 2 * K, 2 * K)])
        return _
    lax.fori_loop(0, ROWS_PER_SUB // (2 * K), _zchunk, None)

    ebase = wid * EDGES_PER_SUB
    pltpu.sync_copy(src_hbm.at[pl.ds(ebase, EDGES_PER_SUB)], ixs_all)
    pltpu.sync_copy(dst_hbm.at[pl.ds(ebase, EDGES_PER_SUB)], ixd_all)
    plsc.subcore_barrier()

    def _fill_idx(u, b):
        vs = ixs_all[pl.ds(u * K, K)]
        vd = ixd_all[pl.ds(u * K, K)]
        igml[b][pl.ds(0, K)] = vs
        igml[b][pl.ds(K, K)] = vd
        icml[b][pl.ds(0, K)] = vd
        icml[b][pl.ds(K, K)] = vs

    def _issue(u, b):
        pltpu.async_copy(t_hbm.at[igml[b]], r_t.at[b], semg[b])
        pltpu.async_copy(
            c_hbm.at[pl.ds(wid * (EDGES_PER_SUB // 2) + u * (K // 2), K // 2)],
            r_c.at[b], semg[b])

    def _drain_g(b):
        pltpu.make_async_copy(t_hbm.at[pl.ds(0, 2 * K)], r_t.at[b], semg[b]).wait()
        pltpu.make_async_copy(c_hbm.at[pl.ds(0, K // 2)], r_c.at[b], semg[b]).wait()

    def _unpk(w):
        lo = lax.bitcast_convert_type(jnp.left_shift(w, 16), jnp.float32)
        hi = lax.bitcast_convert_type(
            jnp.bitwise_and(w, jnp.int32(-65536)), jnp.float32)
        return lo, hi

    def _compute(b):
        @plsc.parallel_loop(0, K // 2, unroll=2)
        def _rowpair(p):
            for o in range(2):
                i = 2 * p + o
                for g in range(D // 32):
                    sla = pl.ds(16 * g, 16)
                    slb = pl.ds(D // 2 + 16 * g, 16)
                    cl, ch = _unpk(r_c[b, p, pl.ds(o * (D // 2) + 16 * g, 16)])
                    asl, ash = _unpk(r_t[b, i, sla])
                    bsl, bsh = _unpk(r_t[b, i, slb])
                    adl, adh = _unpk(r_t[b, K + i, sla])
                    bdl, bdh = _unpk(r_t[b, K + i, slb])
                    x1l = asl + bdl + cl
                    x1h = ash + bdh + ch
                    x2l = adl + bsl + cl
                    x2h = adh + bsh + ch
                    h[b, i, pl.ds(32 * g, 16)] = jnp.maximum(x1l, 0.01 * x1l)
                    h[b, i, pl.ds(32 * g + 16, 16)] = jnp.maximum(x1h, 0.01 * x1h)
                    h[b, K + i, pl.ds(32 * g, 16)] = jnp.maximum(x2l, 0.01 * x2l)
                    h[b, K + i, pl.ds(32 * g + 16, 16)] = jnp.maximum(x2h, 0.01 * x2h)

    def _issue_sc(b):
        pltpu.async_copy(h.at[b], s_sh.at[icml[b]], semsc[b], add=True)

    def _drain_sc(b):
        pltpu.make_async_copy(h.at[b], s_sh.at[icml[b]], semsc[b]).wait()

    _fill_idx(0, 0)
    _issue(0, 0)

    @pl.loop(0, NCHUNK, step=2)
    def _ring(t):
        for b in range(2):
            u = t + b
            nb = (b + 1) % 2

            @pl.when(u < NCHUNK)
            def _():
                @pl.when(u + 1 < NCHUNK)
                def _():
                    @pl.when(u >= 1)
                    def _():
                        _drain_sc(nb)
                    _fill_idx(u + 1, nb)
                    _issue(u + 1, nb)
                _drain_g(b)
                _compute(b)
                _issue_sc(b)

    _drain_sc(0)
    _drain_sc(1)

    plsc.subcore_barrier()
    pltpu.sync_copy(s_sh.at[pl.ds(sid * ROWS_PER_SUB, ROWS_PER_SUB)],
                    out_hbm.at[cid, pl.ds(sid * ROWS_PER_SUB, ROWS_PER_SUB)])


@functools.lru_cache(maxsize=1)
def _edge_call():
    return pl.kernel(
        _edge_body,
        out_type=jax.ShapeDtypeStruct((NC, SPAD, D), jnp.float32),
        mesh=plsc.VectorSubcoreMesh(core_axis_name="c", subcore_axis_name="s"),
        scratch_types=[
        pltpu.VMEM_SHARED((SPAD, D), jnp.float32),
        pltpu.VMEM((EDGES_PER_SUB,), jnp.int32),
        pltpu.VMEM((EDGES_PER_SUB,), jnp.int32),
        pltpu.VMEM((2 * K,), jnp.int32),
        pltpu.VMEM((2 * K,), jnp.int32),
        pltpu.VMEM((2 * K,), jnp.int32),
        pltpu.VMEM((2 * K,), jnp.int32),
        pltpu.VMEM((2, 2 * K, D), jnp.int32),
        pltpu.VMEM((2, K // 2, D), jnp.int32),
        pltpu.VMEM((2, 2 * K, D), jnp.float32),
        pltpu.SemaphoreType.DMA,
        pltpu.SemaphoreType.DMA,
        pltpu.SemaphoreType.DMA,
        pltpu.SemaphoreType.DMA,
        ],
    )


def _rnd_pack(x):
    b = lax.bitcast_convert_type(x, jnp.int32)
    r = b + 0x7FFF + jnp.bitwise_and(lax.shift_right_logical(b, 16), 1)
    lo = lax.shift_right_logical(r[:, :D // 2], 16)
    hi = jnp.bitwise_and(r[:, D // 2:], jnp.int32(-65536))
    return jnp.bitwise_or(hi, lo)


def _prep_t_body(nf_ref, wa_ref, wb_ref, t_ref):
    x = nf_ref[...]
    t_ref[:, :D // 2] = _rnd_pack(x @ wa_ref[...])
    t_ref[:, D // 2:] = _rnd_pack(x @ wb_ref[...])


def _prep_c_body(ef2_ref, wc2_ref, c_ref):
    y = ef2_ref[...] @ wc2_ref[...]
    c_ref[:, :D // 2] = _rnd_pack(y[:, :D])
    c_ref[:, D // 2:] = _rnd_pack(y[:, D:])


def _node_body(sp_ref, nf_ref, we2_ref, wn1a_ref, wn1b_ref, wn2_ref, out_ref):
    red = (sp_ref[0] + sp_ref[1]) @ we2_ref[...]
    x = nf_ref[...] @ wn1a_ref[...] + red @ wn1b_ref[...]
    h = jnp.maximum(x, 0.01 * x)
    out_ref[...] = h @ wn2_ref[...]


_ROW_BLK = 400
_C_BLK = 2000


def kernel(nf, edge_index, ef, W_e1, W_e2, W_n1, W_n2):
    src = edge_index[0]
    dst = edge_index[1]

    T = pl.pallas_call(
        _prep_t_body,
        grid=(N // _ROW_BLK,),
        in_specs=[
            pl.BlockSpec((_ROW_BLK, D), lambda i: (i, 0)),
            pl.BlockSpec((D, D), lambda i: (0, 0)),
            pl.BlockSpec((D, D), lambda i: (0, 0)),
        ],
        out_specs=pl.BlockSpec((_ROW_BLK, D), lambda i: (i, 0)),
        out_shape=jax.ShapeDtypeStruct((N, D), jnp.int32),
    )(nf, W_e1[:D], W_e1[D:2 * D])

    Wc = W_e1[2 * D:]
    Wc2 = jnp.zeros((32, 2 * D), Wc.dtype).at[:16, :D].set(Wc).at[16:, D:].set(Wc)
    C = pl.pallas_call(
        _prep_c_body,
        grid=(E // _C_BLK,),
        in_specs=[
            pl.BlockSpec((_C_BLK // 2, 32), lambda i: (i, 0)),
            pl.BlockSpec((32, 2 * D), lambda i: (0, 0)),
        ],
        out_specs=pl.BlockSpec((_C_BLK // 2, D), lambda i: (i, 0)),
        out_shape=jax.ShapeDtypeStruct((E // 2, D), jnp.int32),
    )(ef.reshape(E // 2, 32), Wc2)

    s_parts = _edge_call()(T, C, src, dst)

    W_e2p = (W_e2.reshape(2, D // 32, 16, D)
             .transpose(1, 0, 2, 3).reshape(D, D))

    return pl.pallas_call(
        _node_body,
        grid=(N // _ROW_BLK,),
        in_specs=[
            pl.BlockSpec((NC, _ROW_BLK, D), lambda i: (0, i, 0)),
            pl.BlockSpec((_ROW_BLK, D), lambda i: (i, 0)),
            pl.BlockSpec((D, D), lambda i: (0, 0)),
            pl.BlockSpec((D, D), lambda i: (0, 0)),
            pl.BlockSpec((D, D), lambda i: (0, 0)),
            pl.BlockSpec((D, D), lambda i: (0, 0)),
        ],
        out_specs=pl.BlockSpec((_ROW_BLK, D), lambda i: (i, 0)),
        out_shape=jax.ShapeDtypeStruct((N, D), jnp.float32),
    )(s_parts, nf, W_e2p, W_n1[:D], W_n1[D:], W_n2)


# speedup vs baseline: 1.0195x; 1.0114x over previous
import functools

import jax
import jax.numpy as jnp
from jax import lax
from jax.experimental import pallas as pl
from jax.experimental.pallas import tpu as pltpu
from jax.experimental.pallas import tpu_sc as plsc

N = 10000
E = 320000
D = 128

NC = 2
NS = 16
NW = NC * NS

SPAD = 10240
ROWS_PER_SUB = SPAD // NS
EDGES_PER_SUB = E // NW
K = 16
NCHUNK = EDGES_PER_SUB // K


def _edge_body(t_hbm, c_hbm, src_hbm, dst_hbm, out_hbm,
               s_sh, ixs_all, ixd_all, ig0, ig1, ic0, ic1,
               r_t, r_c, h,
               semg0, semg1, semsc0, semsc1):
    cid = lax.axis_index("c")
    sid = lax.axis_index("s")
    wid = cid * NS + sid
    semg = (semg0, semg1)
    semsc = (semsc0, semsc1)
    igml = (ig0, ig1)
    icml = (ic0, ic1)

    def _zrow(i, _):
        for j in range(D // 16):
            h[0, i, pl.ds(j * 16, 16)] = jnp.zeros((16,), jnp.float32)
        return _
    lax.fori_loop(0, 2 * K, _zrow, None)

    def _zchunk(t, _):
        pltpu.sync_copy(
            h.at[0], s_sh.at[pl.ds(sid * ROWS_PER_SUB + t * 2 * K, 2 * K)])
        return _
    lax.fori_loop(0, ROWS_PER_SUB // (2 * K), _zchunk, None)

    ebase = wid * EDGES_PER_SUB
    pltpu.sync_copy(src_hbm.at[pl.ds(ebase, EDGES_PER_SUB)], ixs_all)
    pltpu.sync_copy(dst_hbm.at[pl.ds(ebase, EDGES_PER_SUB)], ixd_all)
    plsc.subcore_barrier()

    def _fill_idx(u, b):
        vs = ixs_all[pl.ds(u * K, K)]
        vd = ixd_all[pl.ds(u * K, K)]
        igml[b][pl.ds(0, K)] = vs
        igml[b][pl.ds(K, K)] = vd
        icml[b][pl.ds(0, K)] = vd
        icml[b][pl.ds(K, K)] = vs

    def _issue(u, b):
        pltpu.async_copy(t_hbm.at[igml[b]], r_t.at[b], semg[b])
        pltpu.async_copy(
            c_hbm.at[pl.ds(wid * (EDGES_PER_SUB // 2) + u * (K // 2), K // 2)],
            r_c.at[b], semg[b])

    def _drain_g(b):
        pltpu.make_async_copy(t_hbm.at[pl.ds(0, 2 * K)], r_t.at[b], semg[b]).wait()
        pltpu.make_async_copy(c_hbm.at[pl.ds(0, K // 2)], r_c.at[b], semg[b]).wait()

    def _unpk(w):
        lo = lax.bitcast_convert_type(jnp.left_shift(w, 16), jnp.float32)
        hi = lax.bitcast_convert_type(
            jnp.bitwise_and(w, jnp.int32(-65536)), jnp.float32)
        return lo, hi

    def _compute(b):
        @plsc.parallel_loop(0, K // 2, unroll=2)
        def _rowpair(p):
            for o in range(2):
                i = 2 * p + o
                for g in range(D // 32):
                    sla = pl.ds(16 * g, 16)
                    slb = pl.ds(D // 2 + 16 * g, 16)
                    cl, ch = _unpk(r_c[b, p, pl.ds(o * (D // 2) + 16 * g, 16)])
                    asl, ash = _unpk(r_t[b, i, sla])
                    bsl, bsh = _unpk(r_t[b, i, slb])
                    adl, adh = _unpk(r_t[b, K + i, sla])
                    bdl, bdh = _unpk(r_t[b, K + i, slb])
                    x1l = asl + bdl + cl
                    x1h = ash + bdh + ch
                    x2l = adl + bsl + cl
                    x2h = adh + bsh + ch
                    h[b, i, pl.ds(32 * g, 16)] = jnp.maximum(x1l, 0.01 * x1l)
                    h[b, i, pl.ds(32 * g + 16, 16)] = jnp.maximum(x1h, 0.01 * x1h)
                    h[b, K + i, pl.ds(32 * g, 16)] = jnp.maximum(x2l, 0.01 * x2l)
                    h[b, K + i, pl.ds(32 * g + 16, 16)] = jnp.maximum(x2h, 0.01 * x2h)

    def _issue_sc(b):
        pltpu.async_copy(h.at[b], s_sh.at[icml[b]], semsc[b], add=True)

    def _drain_sc(b):
        pltpu.make_async_copy(h.at[b], s_sh.at[icml[b]], semsc[b]).wait()

    _fill_idx(0, 0)
    _issue(0, 0)

    @pl.loop(0, NCHUNK, step=2)
    def _ring(t):
        for b in range(2):
            u = t + b
            nb = (b + 1) % 2

            @pl.when(u < NCHUNK)
            def _():
                @pl.when(u + 1 < NCHUNK)
                def _():
                    @pl.when(u >= 1)
                    def _():
                        _drain_sc(nb)
                    _fill_idx(u + 1, nb)
                    _issue(u + 1, nb)
                _drain_g(b)
                _compute(b)
                _issue_sc(b)

    _drain_sc(0)
    _drain_sc(1)

    plsc.subcore_barrier()
    pltpu.sync_copy(s_sh.at[pl.ds(sid * ROWS_PER_SUB, ROWS_PER_SUB)],
                    out_hbm.at[cid, pl.ds(sid * ROWS_PER_SUB, ROWS_PER_SUB)])


@functools.lru_cache(maxsize=1)
def _edge_call():
    return pl.kernel(
        _edge_body,
        out_type=jax.ShapeDtypeStruct((NC, SPAD, D), jnp.float32),
        mesh=plsc.VectorSubcoreMesh(core_axis_name="c", subcore_axis_name="s"),
        scratch_types=[
        pltpu.VMEM_SHARED((SPAD, D), jnp.float32),
        pltpu.VMEM((EDGES_PER_SUB,), jnp.int32),
        pltpu.VMEM((EDGES_PER_SUB,), jnp.int32),
        pltpu.VMEM((2 * K,), jnp.int32),
        pltpu.VMEM((2 * K,), jnp.int32),
        pltpu.VMEM((2 * K,), jnp.int32),
        pltpu.VMEM((2 * K,), jnp.int32),
        pltpu.VMEM((2, 2 * K, D), jnp.int32),
        pltpu.VMEM((2, K // 2, D), jnp.int32),
        pltpu.VMEM((2, 2 * K, D), jnp.float32),
        pltpu.SemaphoreType.DMA,
        pltpu.SemaphoreType.DMA,
        pltpu.SemaphoreType.DMA,
        pltpu.SemaphoreType.DMA,
        ],
    )


def _rnd_pack(x):
    b = lax.bitcast_convert_type(x, jnp.int32)
    r = b + 0x7FFF + jnp.bitwise_and(lax.shift_right_logical(b, 16), 1)
    lo = lax.shift_right_logical(r[:, :D // 2], 16)
    hi = jnp.bitwise_and(r[:, D // 2:], jnp.int32(-65536))
    return jnp.bitwise_or(hi, lo)


def _prep_body(nf_ref, wa_ref, wb_ref, ef2_ref, wc2_ref, t_ref, c_ref):
    @pl.when(pl.program_id(0) < N // _ROW_BLK)
    def _():
        x = nf_ref[...]
        t_ref[:, :D // 2] = _rnd_pack(x @ wa_ref[...])
        t_ref[:, D // 2:] = _rnd_pack(x @ wb_ref[...])
    y = ef2_ref[...] @ wc2_ref[...]
    c_ref[:, :D // 2] = _rnd_pack(y[:, :D])
    c_ref[:, D // 2:] = _rnd_pack(y[:, D:])


def _node_body(sp_ref, nf_ref, we2_ref, wn1a_ref, wn1b_ref, wn2_ref, out_ref):
    red = (sp_ref[0] + sp_ref[1]) @ we2_ref[...]
    x = nf_ref[...] @ wn1a_ref[...] + red @ wn1b_ref[...]
    h = jnp.maximum(x, 0.01 * x)
    out_ref[...] = h @ wn2_ref[...]


_ROW_BLK = 400
_C_BLK = 2000


def kernel(nf, edge_index, ef, W_e1, W_e2, W_n1, W_n2):
    src = edge_index[0]
    dst = edge_index[1]

    Wc = W_e1[2 * D:]
    Wc2 = jnp.zeros((32, 2 * D), Wc.dtype).at[:16, :D].set(Wc).at[16:, D:].set(Wc)
    _tclamp = N // _ROW_BLK - 1
    T, C = pl.pallas_call(
        _prep_body,
        grid=(E // _C_BLK,),
        in_specs=[
            pl.BlockSpec((_ROW_BLK, D), lambda i: (jnp.minimum(i, _tclamp), 0)),
            pl.BlockSpec((D, D), lambda i: (0, 0)),
            pl.BlockSpec((D, D), lambda i: (0, 0)),
            pl.BlockSpec((_C_BLK // 2, 32), lambda i: (i, 0)),
            pl.BlockSpec((32, 2 * D), lambda i: (0, 0)),
        ],
        out_specs=[
            pl.BlockSpec((_ROW_BLK, D), lambda i: (jnp.minimum(i, _tclamp), 0)),
            pl.BlockSpec((_C_BLK // 2, D), lambda i: (i, 0)),
        ],
        out_shape=[
            jax.ShapeDtypeStruct((N, D), jnp.int32),
            jax.ShapeDtypeStruct((E // 2, D), jnp.int32),
        ],
    )(nf, W_e1[:D], W_e1[D:2 * D], ef.reshape(E // 2, 32), Wc2)

    s_parts = _edge_call()(T, C, src, dst)

    W_e2p = (W_e2.reshape(2, D // 32, 16, D)
             .transpose(1, 0, 2, 3).reshape(D, D))

    return pl.pallas_call(
        _node_body,
        grid=(N // _ROW_BLK,),
        in_specs=[
            pl.BlockSpec((NC, _ROW_BLK, D), lambda i: (0, i, 0)),
            pl.BlockSpec((_ROW_BLK, D), lambda i: (i, 0)),
            pl.BlockSpec((D, D), lambda i: (0, 0)),
            pl.BlockSpec((D, D), lambda i: (0, 0)),
            pl.BlockSpec((D, D), lambda i: (0, 0)),
            pl.BlockSpec((D, D), lambda i: (0, 0)),
        ],
        out_specs=pl.BlockSpec((_ROW_BLK, D), lambda i: (i, 0)),
        out_shape=jax.ShapeDtypeStruct((N, D), jnp.float32),
    )(s_parts, nf, W_e2p, W_n1[:D], W_n1[D:], W_n2)
